# initial kernel scaffold (unmeasured)
import functools

import jax
import jax.numpy as jnp
from jax import lax
from jax.experimental import pallas as pl
from jax.experimental.pallas import tpu as pltpu

N_DEV = 32


def _a2a_body(x_ref, out_ref, xbf_ref, send_sems, recv_sems, *, m_per, k_shard):
    my = lax.axis_index("i")

    xbf_ref[...] = x_ref[...].astype(jnp.bfloat16)

    for j in range(N_DEV):

        @pl.when(my != j)
        def _(j=j):
            pltpu.make_async_remote_copy(
                src_ref=xbf_ref.at[pl.ds(j * m_per, m_per), :],
                dst_ref=out_ref.at[:, pl.ds(my * k_shard, k_shard)],
                send_sem=send_sems.at[j],
                recv_sem=recv_sems.at[my],
                device_id=(j,),
                device_id_type=pl.DeviceIdType.MESH,
            ).start()

    out_ref[:, pl.ds(my * k_shard, k_shard)] = xbf_ref[pl.ds(my * m_per, m_per), :]

    for j in range(N_DEV):

        @pl.when(my != j)
        def _(j=j):
            desc = pltpu.make_async_remote_copy(
                src_ref=xbf_ref.at[pl.ds(j * m_per, m_per), :],
                dst_ref=out_ref.at[:, pl.ds(j * k_shard, k_shard)],
                send_sem=send_sems.at[j],
                recv_sem=recv_sems.at[j],
                device_id=(j,),
                device_id_type=pl.DeviceIdType.MESH,
            )
            desc.wait_recv()
            desc.wait_send()


def _gemm_body(xg_ref, w_ref, out_ref, *, k_blk, n_k):
    k = pl.program_id(1)

    @pl.when(k == 0)
    def _():
        out_ref[...] = jnp.zeros_like(out_ref)

    wb = w_ref[...].astype(jnp.bfloat16)
    for kk in range(n_k):

        @pl.when(k == kk)
        def _(kk=kk):
            xb = xg_ref[:, kk * k_blk : (kk + 1) * k_blk]
            out_ref[...] += jnp.dot(xb, wb, preferred_element_type=jnp.float32)

    @pl.when(k == n_k - 1)
    def _():
        out_ref[...] = jnp.maximum(out_ref[...], 0.0)


def kernel(x, w_mat):
    m_total, k_shard = x.shape
    k_total, n_total = w_mat.shape
    m_per = m_total // N_DEV

    xg = pl.pallas_call(
        functools.partial(_a2a_body, m_per=m_per, k_shard=k_shard),
        out_shape=jax.ShapeDtypeStruct((m_per, k_total), jnp.bfloat16),
        in_specs=[pl.BlockSpec(memory_space=pltpu.VMEM)],
        out_specs=pl.BlockSpec(memory_space=pltpu.VMEM),
        scratch_shapes=[
            pltpu.VMEM((m_total, k_shard), jnp.bfloat16),
            pltpu.SemaphoreType.DMA((N_DEV,)),
            pltpu.SemaphoreType.DMA((N_DEV,)),
        ],
        compiler_params=pltpu.CompilerParams(collective_id=0),
    )(x)

    n_blk = 512
    k_blk = 2048
    n_k = k_total // k_blk
    grid = (n_total // n_blk, n_k)

    out = pl.pallas_call(
        functools.partial(_gemm_body, k_blk=k_blk, n_k=n_k),
        out_shape=jax.ShapeDtypeStruct((m_per, n_total), jnp.float32),
        grid=grid,
        in_specs=[
            pl.BlockSpec((m_per, k_total), lambda nb, kb: (0, 0)),
            pl.BlockSpec((k_blk, n_blk), lambda nb, kb: (kb, nb)),
        ],
        out_specs=pl.BlockSpec((m_per, n_blk), lambda nb, kb: (0, nb)),
        compiler_params=pltpu.CompilerParams(
            dimension_semantics=("arbitrary", "arbitrary"),
        ),
    )(xg, w_mat)
    return out


# baseline (device time: 126098 ns/iter reference)
import functools

import jax
import jax.numpy as jnp
from jax import lax
from jax.experimental import pallas as pl
from jax.experimental.pallas import tpu as pltpu

N_DEV = 32


def _a2a_body(x_ref, out_ref, xbf_ref, send_sems, recv_sems, *, m_per, k_shard):
    my = lax.axis_index("i")

    xbf_ref[...] = x_ref[...].astype(jnp.bfloat16)

    for j in range(N_DEV):

        @pl.when(my != j)
        def _(j=j):
            pltpu.make_async_remote_copy(
                src_ref=xbf_ref.at[pl.ds(j * m_per, m_per), :],
                dst_ref=out_ref.at[:, pl.ds(my * k_shard, k_shard)],
                send_sem=send_sems.at[j],
                recv_sem=recv_sems.at[my],
                device_id=(j,),
                device_id_type=pl.DeviceIdType.MESH,
            ).start()

    out_ref[:, pl.ds(my * k_shard, k_shard)] = xbf_ref[pl.ds(my * m_per, m_per), :]

    for j in range(N_DEV):

        @pl.when(my != j)
        def _(j=j):
            desc = pltpu.make_async_remote_copy(
                src_ref=xbf_ref.at[pl.ds(j * m_per, m_per), :],
                dst_ref=out_ref.at[:, pl.ds(j * k_shard, k_shard)],
                send_sem=send_sems.at[j],
                recv_sem=recv_sems.at[j],
                device_id=(j,),
                device_id_type=pl.DeviceIdType.MESH,
            )
            desc.wait_recv()
            desc.wait_send()


def _gemm_body(xg_ref, w_ref, out_ref, *, k_blk, n_k):
    k = pl.program_id(1)

    @pl.when(k == 0)
    def _():
        out_ref[...] = jnp.zeros_like(out_ref)

    wb = w_ref[...].astype(jnp.bfloat16)
    for kk in range(n_k):

        @pl.when(k == kk)
        def _(kk=kk):
            xb = xg_ref[:, kk * k_blk : (kk + 1) * k_blk]
            out_ref[...] += jnp.dot(xb, wb, preferred_element_type=jnp.float32)

    @pl.when(k == n_k - 1)
    def _():
        out_ref[...] = jnp.maximum(out_ref[...], 0.0)


def kernel(x, w_mat):
    m_total, k_shard = x.shape
    k_total, n_total = w_mat.shape
    m_per = m_total // N_DEV

    xg = pl.pallas_call(
        functools.partial(_a2a_body, m_per=m_per, k_shard=k_shard),
        out_shape=jax.ShapeDtypeStruct((m_per, k_total), jnp.bfloat16),
        in_specs=[pl.BlockSpec(memory_space=pltpu.VMEM)],
        out_specs=pl.BlockSpec(memory_space=pltpu.VMEM),
        scratch_shapes=[
            pltpu.VMEM((m_total, k_shard), jnp.bfloat16),
            pltpu.SemaphoreType.DMA((N_DEV,)),
            pltpu.SemaphoreType.DMA((N_DEV,)),
        ],
    )(x)

    n_blk = 512
    k_blk = 2048
    n_k = k_total // k_blk
    grid = (n_total // n_blk, n_k)

    out = pl.pallas_call(
        functools.partial(_gemm_body, k_blk=k_blk, n_k=n_k),
        out_shape=jax.ShapeDtypeStruct((m_per, n_total), jnp.float32),
        grid=grid,
        in_specs=[
            pl.BlockSpec((m_per, k_total), lambda nb, kb: (0, 0)),
            pl.BlockSpec((k_blk, n_blk), lambda nb, kb: (kb, nb)),
        ],
        out_specs=pl.BlockSpec((m_per, n_blk), lambda nb, kb: (0, nb)),
        compiler_params=pltpu.CompilerParams(
            dimension_semantics=("arbitrary", "arbitrary"),
        ),
    )(xg, w_mat)
    return out


# device time: 94868 ns/iter; 1.3292x vs baseline; 1.3292x over previous
import functools

import jax
import jax.numpy as jnp
from jax import lax
from jax.experimental import pallas as pl
from jax.experimental.pallas import tpu as pltpu

N_DEV = 32
PLANE = 8


def _body(p_ref, x_ref, w_ref, out_ref, xbf_ref, slots_ref, send_sems, recv_sems,
          *, m_per, k_shard):
    s = pl.program_id(0)
    my = lax.axis_index("i")
    k_eff = (p_ref[0] * PLANE + s) % N_DEV

    @pl.when(s == 0)
    def _():
        out_ref[...] = jnp.zeros_like(out_ref)
        xbf_ref[...] = x_ref[...].astype(jnp.bfloat16)
        slots_ref[my] = xbf_ref[pl.ds(my * m_per, m_per), :]
        for j in range(N_DEV):

            @pl.when(my != j)
            def _(j=j):
                pltpu.make_async_remote_copy(
                    src_ref=xbf_ref.at[pl.ds(j * m_per, m_per), :],
                    dst_ref=slots_ref.at[my],
                    send_sem=send_sems.at[j],
                    recv_sem=recv_sems.at[my],
                    device_id=(j,),
                    device_id_type=pl.DeviceIdType.MESH,
                ).start()

    @pl.when(k_eff != my)
    def _():
        pltpu.make_async_remote_copy(
            src_ref=xbf_ref.at[pl.ds(0, m_per), :],
            dst_ref=slots_ref.at[k_eff],
            send_sem=send_sems.at[0],
            recv_sem=recv_sems.at[k_eff],
            device_id=(0,),
            device_id_type=pl.DeviceIdType.MESH,
        ).wait_recv()

    out_ref[...] += jnp.dot(
        slots_ref[k_eff],
        w_ref[...].astype(jnp.bfloat16),
        preferred_element_type=jnp.float32,
    )

    @pl.when(s == N_DEV - 1)
    def _():
        out_ref[...] = jnp.maximum(out_ref[...], 0.0)
        for j in range(N_DEV):

            @pl.when(my != j)
            def _(j=j):
                pltpu.make_async_remote_copy(
                    src_ref=xbf_ref.at[pl.ds(j * m_per, m_per), :],
                    dst_ref=slots_ref.at[0],
                    send_sem=send_sems.at[j],
                    recv_sem=recv_sems.at[0],
                    device_id=(j,),
                    device_id_type=pl.DeviceIdType.MESH,
                ).wait_send()


def kernel(x, w_mat):
    m_total, k_shard = x.shape
    k_total, n_total = w_mat.shape
    m_per = m_total // N_DEV

    plane = jnp.reshape((lax.axis_index("i") // PLANE).astype(jnp.int32), (1,))

    grid_spec = pltpu.PrefetchScalarGridSpec(
        num_scalar_prefetch=1,
        grid=(N_DEV,),
        in_specs=[
            pl.BlockSpec((m_total, k_shard), lambda s, p: (0, 0)),
            pl.BlockSpec(
                (k_shard, n_total),
                lambda s, p: ((p[0] * PLANE + s) % N_DEV, 0),
            ),
        ],
        out_specs=pl.BlockSpec((m_per, n_total), lambda s, p: (0, 0)),
        scratch_shapes=[
            pltpu.VMEM((m_total, k_shard), jnp.bfloat16),
            pltpu.VMEM((N_DEV, m_per, k_shard), jnp.bfloat16),
            pltpu.SemaphoreType.DMA((N_DEV,)),
            pltpu.SemaphoreType.DMA((N_DEV,)),
        ],
    )

    return pl.pallas_call(
        functools.partial(_body, m_per=m_per, k_shard=k_shard),
        grid_spec=grid_spec,
        out_shape=jax.ShapeDtypeStruct((m_per, n_total), jnp.float32),
        compiler_params=pltpu.CompilerParams(
            dimension_semantics=("arbitrary",),
        ),
    )(plane, x, w_mat)


# device time: 89177 ns/iter; 1.4140x vs baseline; 1.0638x over previous
import functools

import jax
import jax.numpy as jnp
import numpy as np
from jax import lax
from jax.experimental import pallas as pl
from jax.experimental.pallas import tpu as pltpu

N_DEV = 32
PLANE = 8


def _pos_coords(p):
    z, q = divmod(p, PLANE)
    y, r = divmod(q, 2)
    x = r if y % 2 == 0 else 1 - r
    return x, y, z


_COORDS = [_pos_coords(p) for p in range(N_DEV)]
_ORDER = np.array(
    [
        sorted(
            range(N_DEV),
            key=lambda j: (
                abs(ci[0] - _COORDS[j][0])
                + abs(ci[1] - _COORDS[j][1])
                + abs(ci[2] - _COORDS[j][2]),
                j,
            ),
        )
        for ci in _COORDS
    ],
    dtype=np.int32,
)


def _body(order_ref, x_ref, w_ref, out_ref, xbf_ref, slots_ref, wbuf_ref,
          send_sems, recv_sems, wsems, *, m_per, k_shard, n_total):
    my = lax.axis_index("i")

    def w_chunk_dma(s_idx, buf):
        k_eff = order_ref[s_idx]
        return pltpu.make_async_copy(
            w_ref.at[pl.ds(k_eff * k_shard, k_shard), :],
            wbuf_ref.at[buf],
            wsems.at[buf],
        )

    w_chunk_dma(0, 0).start()

    xbf_ref[...] = x_ref[...].astype(jnp.bfloat16)
    slots_ref[my] = xbf_ref[pl.ds(my * m_per, m_per), :]

    def send(jj, _):
        dst = order_ref[jj]
        pltpu.make_async_remote_copy(
            src_ref=xbf_ref.at[pl.ds(dst * m_per, m_per), :],
            dst_ref=slots_ref.at[my],
            send_sem=send_sems.at[dst],
            recv_sem=recv_sems.at[my],
            device_id=(dst,),
            device_id_type=pl.DeviceIdType.MESH,
        ).start()
        return 0

    lax.fori_loop(1, N_DEV, send, 0)

    out_ref[...] = jnp.zeros_like(out_ref)

    def step(s_idx, buf):
        k_eff = order_ref[s_idx]
        w_chunk_dma(s_idx, buf).wait()

        @pl.when(k_eff != my)
        def _():
            pltpu.make_async_remote_copy(
                src_ref=xbf_ref.at[pl.ds(0, m_per), :],
                dst_ref=slots_ref.at[k_eff],
                send_sem=send_sems.at[my],
                recv_sem=recv_sems.at[k_eff],
                device_id=(my,),
                device_id_type=pl.DeviceIdType.MESH,
            ).wait_recv()

        out_ref[...] += jnp.dot(
            slots_ref[k_eff],
            wbuf_ref[buf].astype(jnp.bfloat16),
            preferred_element_type=jnp.float32,
        )

    def pair(i, _):
        s0 = 2 * i
        w_chunk_dma(s0 + 1, 1).start()
        step(s0, 0)
        @pl.when(s0 + 2 < N_DEV)
        def _():
            w_chunk_dma(s0 + 2, 0).start()

        step(s0 + 1, 1)
        return 0

    lax.fori_loop(0, N_DEV // 2, pair, 0)

    out_ref[...] = jnp.maximum(out_ref[...], 0.0)

    def drain(jj, _):
        dst = order_ref[jj]
        pltpu.make_async_remote_copy(
            src_ref=xbf_ref.at[pl.ds(0, m_per), :],
            dst_ref=slots_ref.at[0],
            send_sem=send_sems.at[dst],
            recv_sem=recv_sems.at[0],
            device_id=(dst,),
            device_id_type=pl.DeviceIdType.MESH,
        ).wait_send()
        return 0

    lax.fori_loop(1, N_DEV, drain, 0)


def kernel(x, w_mat):
    m_total, k_shard = x.shape
    k_total, n_total = w_mat.shape
    m_per = m_total // N_DEV

    order = jnp.take(jnp.asarray(_ORDER), lax.axis_index("i"), axis=0)

    return pl.pallas_call(
        functools.partial(_body, m_per=m_per, k_shard=k_shard, n_total=n_total),
        out_shape=jax.ShapeDtypeStruct((m_per, n_total), jnp.float32),
        in_specs=[
            pl.BlockSpec(memory_space=pltpu.SMEM),
            pl.BlockSpec(memory_space=pltpu.VMEM),
            pl.BlockSpec(memory_space=pl.ANY),
        ],
        out_specs=pl.BlockSpec(memory_space=pltpu.VMEM),
        scratch_shapes=[
            pltpu.VMEM((m_total, k_shard), jnp.bfloat16),
            pltpu.VMEM((N_DEV, m_per, k_shard), jnp.bfloat16),
            pltpu.VMEM((2, k_shard, n_total), jnp.float32),
            pltpu.SemaphoreType.DMA((N_DEV,)),
            pltpu.SemaphoreType.DMA((N_DEV,)),
            pltpu.SemaphoreType.DMA((2,)),
        ],
    )(order, x, w_mat)


# device time: 83458 ns/iter; 1.5109x vs baseline; 1.0685x over previous
import functools

import jax
import jax.numpy as jnp
import numpy as np
from jax import lax
from jax.experimental import pallas as pl
from jax.experimental.pallas import tpu as pltpu

N_DEV = 32
PLANE = 8


def _pos_coords(p):
    z, q = divmod(p, PLANE)
    y, r = divmod(q, 2)
    x = r if y % 2 == 0 else 1 - r
    return x, y, z


_COORDS = [_pos_coords(p) for p in range(N_DEV)]
_ORDER = np.array(
    [
        sorted(
            range(N_DEV),
            key=lambda j: (
                abs(ci[0] - _COORDS[j][0])
                + abs(ci[1] - _COORDS[j][1])
                + abs(ci[2] - _COORDS[j][2]),
                j,
            ),
        )
        for ci in _COORDS
    ],
    dtype=np.int32,
)


def _body(order_ref, x_ref, w_ref, out_ref, xbf_ref, slots_ref, wbuf_ref,
          send_sems, recv_sems, wsems, *, m_per, k_shard, n_total):
    my = lax.axis_index("i")

    def w_chunk_dma(s_idx, buf):
        k_eff = order_ref[s_idx]
        return pltpu.make_async_copy(
            w_ref.at[pl.ds(k_eff * k_shard, k_shard), :],
            wbuf_ref.at[buf],
            wsems.at[buf],
        )

    w_chunk_dma(0, 0).start()
    w_chunk_dma(1, 1).start()
    w_chunk_dma(2, 2).start()

    xbf_ref[...] = x_ref[...].astype(jnp.bfloat16)
    slots_ref[my] = xbf_ref[pl.ds(my * m_per, m_per), :]

    def send(jj, _):
        dst = order_ref[jj]
        pltpu.make_async_remote_copy(
            src_ref=xbf_ref.at[pl.ds(dst * m_per, m_per), :],
            dst_ref=slots_ref.at[my],
            send_sem=send_sems.at[dst],
            recv_sem=recv_sems.at[my],
            device_id=(dst,),
            device_id_type=pl.DeviceIdType.MESH,
        ).start()
        return 0

    lax.fori_loop(1, N_DEV, send, 0)

    out_ref[...] = jnp.zeros_like(out_ref)

    def step(s_idx, buf):
        k_eff = order_ref[s_idx]
        w_chunk_dma(s_idx, buf).wait()

        @pl.when(k_eff != my)
        def _():
            pltpu.make_async_remote_copy(
                src_ref=xbf_ref.at[pl.ds(0, m_per), :],
                dst_ref=slots_ref.at[k_eff],
                send_sem=send_sems.at[my],
                recv_sem=recv_sems.at[k_eff],
                device_id=(my,),
                device_id_type=pl.DeviceIdType.MESH,
            ).wait_recv()

        out_ref[...] += jnp.dot(
            slots_ref[k_eff],
            wbuf_ref[buf].astype(jnp.bfloat16),
            preferred_element_type=jnp.float32,
        )

    def quad(i, _):
        s0 = 4 * i
        for par in range(4):
            s = s0 + par

            @pl.when(s + 3 < N_DEV)
            def _(s=s, par=par):
                w_chunk_dma(s + 3, (par + 3) % 4).start()

            step(s, par)
        return 0

    lax.fori_loop(0, N_DEV // 4, quad, 0)

    out_ref[...] = jnp.maximum(out_ref[...], 0.0)

    def drain(jj, _):
        dst = order_ref[jj]
        pltpu.make_async_remote_copy(
            src_ref=xbf_ref.at[pl.ds(0, m_per), :],
            dst_ref=slots_ref.at[0],
            send_sem=send_sems.at[dst],
            recv_sem=recv_sems.at[0],
            device_id=(dst,),
            device_id_type=pl.DeviceIdType.MESH,
        ).wait_send()
        return 0

    lax.fori_loop(1, N_DEV, drain, 0)


def kernel(x, w_mat):
    m_total, k_shard = x.shape
    k_total, n_total = w_mat.shape
    m_per = m_total // N_DEV

    order = jnp.take(jnp.asarray(_ORDER), lax.axis_index("i"), axis=0)

    return pl.pallas_call(
        functools.partial(_body, m_per=m_per, k_shard=k_shard, n_total=n_total),
        out_shape=jax.ShapeDtypeStruct((m_per, n_total), jnp.float32),
        in_specs=[
            pl.BlockSpec(memory_space=pltpu.SMEM),
            pl.BlockSpec(memory_space=pltpu.VMEM),
            pl.BlockSpec(memory_space=pl.ANY),
        ],
        out_specs=pl.BlockSpec(memory_space=pltpu.VMEM),
        scratch_shapes=[
            pltpu.VMEM((m_total, k_shard), jnp.bfloat16),
            pltpu.VMEM((N_DEV, m_per, k_shard), jnp.bfloat16),
            pltpu.VMEM((4, k_shard, n_total), jnp.float32),
            pltpu.SemaphoreType.DMA((N_DEV,)),
            pltpu.SemaphoreType.DMA((N_DEV,)),
            pltpu.SemaphoreType.DMA((4,)),
        ],
    )(order, x, w_mat)
